# baseline (speedup 1.0000x reference)
import functools

import jax
import jax.numpy as jnp
from jax import lax
from jax.experimental import pallas as pl
from jax.experimental.pallas import tpu as pltpu
from jax.experimental.pallas import tpu_sc as plsc

_D = 64
_NB = 100000
_NA = 1024
_BUCKETS = 1024
_CAP = 256
_NC = 2
_NS = 16
_NW = _NC * _NS
_RPW = 3200
_NPAD = _RPW * _NW
_NGRP = _RPW // 16
_HSIZE = 1040
_ACC_ROWS = 1152
_CHUNKS = _RPW // 128


def _iota16():
    return lax.broadcasted_iota(jnp.int32, (16,), 0)


def _sc1_body(keys_hbm, h_out, lp_out, hist_out,
              keys_v, h_v, lp_v, hist_v, sbufp, sbufn):
    c = lax.axis_index("c")
    s = lax.axis_index("s")
    w = s * _NC + c
    base_row = w * _RPW
    pltpu.sync_copy(keys_hbm.at[pl.ds(base_row * 4, _RPW * 4)], keys_v)
    iota = _iota16()
    zero = jnp.zeros((16,), jnp.int32)

    def zb(i, carry):
        hist_v[pl.ds(i * 16, 16)] = zero
        return carry

    lax.fori_loop(0, _HSIZE // 16, zb, 0)

    def grp(g, carry):
        row0 = g * 16
        ridx = row0 + iota
        ridx4 = ridx * 4
        k = plsc.load_gather(keys_v, [ridx4])
        for col in (1, 2, 3):
            k = k + plsc.load_gather(keys_v, [ridx4 + col])
        h = jnp.bitwise_and(k, _BUCKETS - 1)
        h = jnp.where(base_row + ridx < _NB, h, _BUCKETS)
        h_v[pl.ds(row0, 16)] = h
        hs, lid = plsc.sort_key_val(h, iota)
        sbufp[pl.ds(0, 16)] = hs
        sbufp[pl.ds(1, 16)] = hs
        prev = sbufp[pl.ds(0, 16)]
        sbufn[pl.ds(1, 16)] = hs
        sbufn[pl.ds(0, 16)] = hs
        nxt = sbufn[pl.ds(1, 16)]
        is_start = (hs != prev) | (iota == 0)
        is_last = (hs != nxt) | (iota == 15)
        start_pos = plsc.cummax(jnp.where(is_start, iota, 0))
        rank = iota - start_pos
        cur = plsc.load_gather(hist_v, [hs])
        lp_sorted = cur + rank
        plsc.store_scatter(hist_v, [hs], lp_sorted + 1, mask=is_last)
        plsc.store_scatter(lp_v, [row0 + lid], lp_sorted)
        return carry

    lax.fori_loop(0, _NGRP, grp, 0)
    pltpu.sync_copy(h_v, h_out.at[w])
    pltpu.sync_copy(lp_v, lp_out.at[w])
    pltpu.sync_copy(hist_v, hist_out.at[w])


_HD = _D // 2
_NSLAB = 1


_SLICE = _NA // _NS


def _sc2_body(h_hbm, lp_hbm, hist_hbm, emb_hbm, partial_out,
              hrow_v, base_v, h_v, lp_v, ebuf, acc_v, red_v, tmp_v, stage):
    c = lax.axis_index("c")
    s = lax.axis_index("s")
    w = s * _NC + c
    base_row = w * _RPW
    pltpu.sync_copy(h_hbm.at[w], h_v)
    pltpu.sync_copy(lp_hbm.at[w], lp_v)
    zero = jnp.zeros((16,), jnp.int32)
    zf = jnp.zeros((16,), jnp.float32)

    def zb(i, carry):
        base_v[pl.ds(i * 16, 16)] = zero
        return carry

    lax.fori_loop(0, _HSIZE // 16, zb, 0)

    def srow(wp, carry):
        pltpu.sync_copy(hist_hbm.at[wp], hrow_v)

        def sc(i, carry2):
            base_v[pl.ds(i * 16, 16)] = (
                base_v[pl.ds(i * 16, 16)] + hrow_v[pl.ds(i * 16, 16)])
            return carry2
        return lax.fori_loop(0, _HSIZE // 16, sc, carry)

    lax.fori_loop(0, w, srow, 0)

    def grp(g, carry):
        row0 = g * 16
        h = h_v[pl.ds(row0, 16)]
        lpv = lp_v[pl.ds(row0, 16)]
        b = plsc.load_gather(base_v, [h])
        h_v[pl.ds(row0, 16)] = jnp.where(b + lpv < _CAP, h, _BUCKETS)
        return carry

    lax.fori_loop(0, _NGRP, grp, 0)

    for half in range(2):
        def zacc(i, carry):
            acc_v[pl.ds(i * 16, 16)] = zf
            return carry

        lax.fori_loop(0, _HSIZE * _HD // 16, zacc, 0)

        def chunk_body(j, carry):
            row_b = base_row + j * 128
            nrows = jnp.maximum(jnp.minimum(_NB - row_b, 128), 0)

            @pl.when(nrows == 128)
            def _full():
                pltpu.sync_copy(emb_hbm.at[pl.ds(row_b * _D, 128 * _D)], ebuf)

            @pl.when((nrows > 0) & (nrows < 128))
            def _tail():
                pltpu.sync_copy(
                    emb_hbm.at[pl.ds(row_b * _D, (_NB % 128) * _D)],
                    ebuf.at[pl.ds(0, (_NB % 128) * _D)])

            def grp_add(g, jj):
                tgt16 = h_v[pl.ds(jj * 128 + g * 16, 16)]
                for l in range(16):
                    a0 = tgt16[l] * _HD
                    e0 = (g * 16 + l) * _D + half * _HD
                    for col in range(_HD // 16):
                        acc_v[pl.ds(a0 + col * 16, 16)] = (
                            acc_v[pl.ds(a0 + col * 16, 16)]
                            + ebuf[pl.ds(e0 + col * 16, 16)])
                return jj

            lax.fori_loop(0, nrows // 16, grp_add, j)
            return carry

        lax.fori_loop(0, _CHUNKS, chunk_body, 0)

        r0 = s * _SLICE * _HD

        def zred(i, carry):
            red_v[pl.ds(i * 16, 16)] = zf
            return carry

        lax.fori_loop(0, _SLICE * _HD // 16, zred, 0)

        def wave_body(wave, carry):
            @pl.when(s == wave)
            def _pub():
                pltpu.sync_copy(acc_v, stage)

            plsc.subcore_barrier()
            pltpu.sync_copy(stage.at[pl.ds(r0, _SLICE * _HD)], tmp_v)

            def radd(i, carry2):
                sl = pl.ds(i * 16, 16)
                red_v[sl] = red_v[sl] + tmp_v[sl]
                return carry2

            lax.fori_loop(0, _SLICE * _HD // 16, radd, 0)
            plsc.subcore_barrier()
            return carry

        lax.fori_loop(0, _NS, wave_body, 0)

        pltpu.sync_copy(red_v,
                        partial_out.at[half, c, pl.ds(r0, _SLICE * _HD)])


def _mlp_kernel(ta_ref, ka_ref, part_ref, hist_ref,
                w1_ref, b1_ref, w2_ref, b2_ref, out_ref):
    ha = jnp.sum(ka_ref[...], axis=1) % _BUCKETS
    iota_b = lax.broadcasted_iota(jnp.int32, (_NA, _BUCKETS), 1)
    oh = (ha[:, None] == iota_b).astype(jnp.float32)
    bsum = jnp.concatenate(
        [part_ref[0, 0] + part_ref[0, 1],
         part_ref[1, 0] + part_ref[1, 1]], axis=1)
    agg_sum = jnp.dot(oh, bsum, preferred_element_type=jnp.float32)
    hist_f = hist_ref[...][:, :_BUCKETS].astype(jnp.float32)
    cnt_b = jnp.sum(hist_f, axis=0, keepdims=True)
    cnt_a = lax.dot_general(
        oh, cnt_b, (((1,), (1,)), ((), ())),
        preferred_element_type=jnp.float32)
    scale = 1.0 / jnp.maximum(jnp.minimum(cnt_a, float(_CAP)), 1.0)
    agg = agg_sum * scale
    w1 = w1_ref[...]
    x = (
        jnp.dot(ta_ref[...], w1[:_D], preferred_element_type=jnp.float32)
        + jnp.dot(agg, w1[_D:], preferred_element_type=jnp.float32)
        + b1_ref[...]
    )
    x = jnp.maximum(x, 0.0)
    out_ref[...] = (
        jnp.dot(x, w2_ref[...], preferred_element_type=jnp.float32)
        + b2_ref[...]
    )


def kernel(table_a_emb, table_b_emb, keys_a, keys_b, W1, b1, W2, b2):
    mesh = plsc.VectorSubcoreMesh(core_axis_name="c", subcore_axis_name="s")

    sc_params = pltpu.CompilerParams(needs_layout_passes=False)
    sc1 = functools.partial(
        pl.kernel,
        mesh=mesh,
        compiler_params=sc_params,
        out_type=[
            jax.ShapeDtypeStruct((_NW, _RPW), jnp.int32),
            jax.ShapeDtypeStruct((_NW, _RPW), jnp.int32),
            jax.ShapeDtypeStruct((_NW, _HSIZE), jnp.int32),
        ],
        scratch_types=[
            pltpu.VMEM((_RPW * 4,), jnp.int32),
            pltpu.VMEM((_RPW,), jnp.int32),
            pltpu.VMEM((_RPW,), jnp.int32),
            pltpu.VMEM((_HSIZE,), jnp.int32),
            pltpu.VMEM((32,), jnp.int32),
            pltpu.VMEM((32,), jnp.int32),
        ],
    )(_sc1_body)

    keys_pad = jnp.pad(keys_b, ((0, _NPAD - _NB), (0, 0))).reshape(_NPAD * 4)
    h_all, lp_all, hist = sc1(keys_pad)

    sc2 = functools.partial(
        pl.kernel,
        mesh=mesh,
        compiler_params=sc_params,
        out_type=[
            jax.ShapeDtypeStruct((2, _NC, _NA * _HD), jnp.float32),
        ],
        scratch_types=[
            pltpu.VMEM((_HSIZE,), jnp.int32),
            pltpu.VMEM((_HSIZE,), jnp.int32),
            pltpu.VMEM((_RPW,), jnp.int32),
            pltpu.VMEM((_RPW,), jnp.int32),
            pltpu.VMEM((128 * _D,), jnp.float32),
            pltpu.VMEM((_HSIZE * _HD,), jnp.float32),
            pltpu.VMEM((_SLICE * _HD,), jnp.float32),
            pltpu.VMEM((_SLICE * _HD,), jnp.float32),
            pltpu.VMEM_SHARED((_HSIZE * _HD,), jnp.float32),
        ],
    )(_sc2_body)

    (partial,) = sc2(h_all, lp_all, hist, table_b_emb.reshape(_NB * _D))
    partial = partial.reshape(2, _NC, _NA, _HD)

    out = pl.pallas_call(
        _mlp_kernel,
        out_shape=jax.ShapeDtypeStruct((_NA, _D), jnp.float32),
    )(
        table_a_emb,
        keys_a,
        partial,
        hist,
        W1,
        b1.reshape(1, -1),
        W2,
        b2.reshape(1, -1),
    )
    return out


# speedup vs baseline: 1.1105x; 1.1105x over previous
import functools

import jax
import jax.numpy as jnp
from jax import lax
from jax.experimental import pallas as pl
from jax.experimental.pallas import tpu as pltpu
from jax.experimental.pallas import tpu_sc as plsc

_D = 64
_NB = 100000
_NA = 1024
_BUCKETS = 1024
_CAP = 256
_NC = 2
_NS = 16
_NW = _NC * _NS
_RPW = 3200
_NPAD = _RPW * _NW
_NGRP = _RPW // 16
_HSIZE = 1040
_CROWS = 320
_CHUNKS = _RPW // _CROWS


def _iota16():
    return lax.broadcasted_iota(jnp.int32, (16,), 0)


def _sc1_body(keys_hbm, h_out, lp_out, hist_out,
              keys_v, h_v, lp_v, hist_v, sbufp, sbufn):
    c = lax.axis_index("c")
    s = lax.axis_index("s")
    w = s * _NC + c
    base_row = w * _RPW
    pltpu.sync_copy(keys_hbm.at[pl.ds(base_row * 4, _RPW * 4)], keys_v)
    iota = _iota16()
    zero = jnp.zeros((16,), jnp.int32)

    def zb(i, carry):
        hist_v[pl.ds(i * 16, 16)] = zero
        return carry

    lax.fori_loop(0, _HSIZE // 16, zb, 0)

    def grp(g, carry):
        row0 = g * 16
        ridx = row0 + iota
        ridx4 = ridx * 4
        k = plsc.load_gather(keys_v, [ridx4])
        for col in (1, 2, 3):
            k = k + plsc.load_gather(keys_v, [ridx4 + col])
        h = jnp.bitwise_and(k, _BUCKETS - 1)
        h = jnp.where(base_row + ridx < _NB, h, _BUCKETS)
        h_v[pl.ds(row0, 16)] = h
        hs, lid = plsc.sort_key_val(h, iota)
        sbufp[pl.ds(0, 16)] = hs
        sbufp[pl.ds(1, 16)] = hs
        prev = sbufp[pl.ds(0, 16)]
        sbufn[pl.ds(1, 16)] = hs
        sbufn[pl.ds(0, 16)] = hs
        nxt = sbufn[pl.ds(1, 16)]
        is_start = (hs != prev) | (iota == 0)
        is_last = (hs != nxt) | (iota == 15)
        start_pos = plsc.cummax(jnp.where(is_start, iota, 0))
        rank = iota - start_pos
        cur = plsc.load_gather(hist_v, [hs])
        lp_sorted = cur + rank
        plsc.store_scatter(hist_v, [hs], lp_sorted + 1, mask=is_last)
        plsc.store_scatter(lp_v, [row0 + lid], lp_sorted)
        return carry

    lax.fori_loop(0, _NGRP, grp, 0)
    pltpu.sync_copy(h_v, h_out.at[w])
    pltpu.sync_copy(lp_v, lp_out.at[w])
    pltpu.sync_copy(hist_v, hist_out.at[w])


_HD = _D // 2
_NSLAB = 1


_SLICE = _NA // _NS


def _sc2_body(h_hbm, lp_hbm, base_hbm, emb_hbm, partial_out,
              base_v, h_v, lp_v, ebuf, acc_v, red_v, tmp_v, stage):
    c = lax.axis_index("c")
    s = lax.axis_index("s")
    w = s * _NC + c
    base_row = w * _RPW
    pltpu.sync_copy(h_hbm.at[w], h_v)
    pltpu.sync_copy(lp_hbm.at[w], lp_v)
    pltpu.sync_copy(base_hbm.at[w], base_v)
    zf = jnp.zeros((16,), jnp.float32)

    def grp(g, carry):
        row0 = g * 16
        h = h_v[pl.ds(row0, 16)]
        lpv = lp_v[pl.ds(row0, 16)]
        b = plsc.load_gather(base_v, [h])
        h_v[pl.ds(row0, 16)] = jnp.where(b + lpv < _CAP, h, _BUCKETS)
        return carry

    lax.fori_loop(0, _NGRP, grp, 0)

    for half in range(2):
        def zacc(i, carry):
            acc_v[pl.ds(i * 16, 16)] = zf
            return carry

        lax.fori_loop(0, _HSIZE * _HD // 16, zacc, 0)

        def chunk_body(j, carry):
            row_b = base_row + j * _CROWS
            nrows = jnp.maximum(jnp.minimum(_NB - row_b, _CROWS), 0)

            @pl.when(nrows == _CROWS)
            def _full():
                pltpu.sync_copy(
                    emb_hbm.at[pl.ds(row_b * _D, _CROWS * _D)], ebuf)

            @pl.when((nrows > 0) & (nrows < _CROWS))
            def _tail():
                pltpu.sync_copy(
                    emb_hbm.at[pl.ds(row_b * _D, (_CROWS // 2) * _D)],
                    ebuf.at[pl.ds(0, (_CROWS // 2) * _D)])

            def grp_add(g, jj):
                tgt16 = h_v[pl.ds(jj * _CROWS + g * 16, 16)]
                for l in range(16):
                    a0 = tgt16[l] * _HD
                    e0 = (g * 16 + l) * _D + half * _HD
                    for col in range(_HD // 16):
                        acc_v[pl.ds(a0 + col * 16, 16)] = (
                            acc_v[pl.ds(a0 + col * 16, 16)]
                            + ebuf[pl.ds(e0 + col * 16, 16)])
                return jj

            lax.fori_loop(0, nrows // 16, grp_add, j)
            return carry

        lax.fori_loop(0, _CHUNKS, chunk_body, 0)

        r0 = s * _SLICE * _HD

        def zred(i, carry):
            red_v[pl.ds(i * 16, 16)] = zf
            return carry

        lax.fori_loop(0, _SLICE * _HD // 16, zred, 0)

        def wave_body(wave, carry):
            @pl.when(s == wave)
            def _pub():
                pltpu.sync_copy(acc_v, stage)

            plsc.subcore_barrier()
            pltpu.sync_copy(stage.at[pl.ds(r0, _SLICE * _HD)], tmp_v)

            def radd(i, carry2):
                sl = pl.ds(i * 16, 16)
                red_v[sl] = red_v[sl] + tmp_v[sl]
                return carry2

            lax.fori_loop(0, _SLICE * _HD // 16, radd, 0)
            plsc.subcore_barrier()
            return carry

        lax.fori_loop(0, _NS, wave_body, 0)

        pltpu.sync_copy(red_v,
                        partial_out.at[half, c, pl.ds(r0, _SLICE * _HD)])


def _prefix_kernel(hist_ref, out_ref):
    x = hist_ref[...].astype(jnp.float32)
    tri = (
        jax.lax.broadcasted_iota(jnp.int32, (_NW, _NW), 0)
        > jax.lax.broadcasted_iota(jnp.int32, (_NW, _NW), 1)
    ).astype(jnp.float32)
    out_ref[...] = jnp.dot(
        tri, x, preferred_element_type=jnp.float32).astype(jnp.int32)


def _mlp_kernel(ta_ref, ka_ref, part_ref, hist_ref,
                w1_ref, b1_ref, w2_ref, b2_ref, out_ref):
    ha = jnp.sum(ka_ref[...], axis=1) % _BUCKETS
    iota_b = lax.broadcasted_iota(jnp.int32, (_NA, _BUCKETS), 1)
    oh = (ha[:, None] == iota_b).astype(jnp.float32)
    bsum = jnp.concatenate(
        [part_ref[0, 0] + part_ref[0, 1],
         part_ref[1, 0] + part_ref[1, 1]], axis=1)
    agg_sum = jnp.dot(oh, bsum, preferred_element_type=jnp.float32)
    hist_f = hist_ref[...][:, :_BUCKETS].astype(jnp.float32)
    cnt_b = jnp.sum(hist_f, axis=0, keepdims=True)
    cnt_a = lax.dot_general(
        oh, cnt_b, (((1,), (1,)), ((), ())),
        preferred_element_type=jnp.float32)
    scale = 1.0 / jnp.maximum(jnp.minimum(cnt_a, float(_CAP)), 1.0)
    agg = agg_sum * scale
    w1 = w1_ref[...]
    x = (
        jnp.dot(ta_ref[...], w1[:_D], preferred_element_type=jnp.float32)
        + jnp.dot(agg, w1[_D:], preferred_element_type=jnp.float32)
        + b1_ref[...]
    )
    x = jnp.maximum(x, 0.0)
    out_ref[...] = (
        jnp.dot(x, w2_ref[...], preferred_element_type=jnp.float32)
        + b2_ref[...]
    )


def kernel(table_a_emb, table_b_emb, keys_a, keys_b, W1, b1, W2, b2):
    mesh = plsc.VectorSubcoreMesh(core_axis_name="c", subcore_axis_name="s")

    sc_params = pltpu.CompilerParams(needs_layout_passes=False)
    sc1 = functools.partial(
        pl.kernel,
        mesh=mesh,
        compiler_params=sc_params,
        out_type=[
            jax.ShapeDtypeStruct((_NW, _RPW), jnp.int32),
            jax.ShapeDtypeStruct((_NW, _RPW), jnp.int32),
            jax.ShapeDtypeStruct((_NW, _HSIZE), jnp.int32),
        ],
        scratch_types=[
            pltpu.VMEM((_RPW * 4,), jnp.int32),
            pltpu.VMEM((_RPW,), jnp.int32),
            pltpu.VMEM((_RPW,), jnp.int32),
            pltpu.VMEM((_HSIZE,), jnp.int32),
            pltpu.VMEM((32,), jnp.int32),
            pltpu.VMEM((32,), jnp.int32),
        ],
    )(_sc1_body)

    keys_pad = jnp.pad(keys_b, ((0, _NPAD - _NB), (0, 0))).reshape(_NPAD * 4)
    h_all, lp_all, hist = sc1(keys_pad)

    base = pl.pallas_call(
        _prefix_kernel,
        out_shape=jax.ShapeDtypeStruct((_NW, _HSIZE), jnp.int32),
    )(hist)

    sc2 = functools.partial(
        pl.kernel,
        mesh=mesh,
        compiler_params=sc_params,
        out_type=[
            jax.ShapeDtypeStruct((2, _NC, _NA * _HD), jnp.float32),
        ],
        scratch_types=[
            pltpu.VMEM((_HSIZE,), jnp.int32),
            pltpu.VMEM((_RPW,), jnp.int32),
            pltpu.VMEM((_RPW,), jnp.int32),
            pltpu.VMEM((_CROWS * _D,), jnp.float32),
            pltpu.VMEM((_HSIZE * _HD,), jnp.float32),
            pltpu.VMEM((_SLICE * _HD,), jnp.float32),
            pltpu.VMEM((_SLICE * _HD,), jnp.float32),
            pltpu.VMEM_SHARED((_HSIZE * _HD,), jnp.float32),
        ],
    )(_sc2_body)

    (partial,) = sc2(h_all, lp_all, base, table_b_emb.reshape(_NB * _D))
    partial = partial.reshape(2, _NC, _NA, _HD)

    out = pl.pallas_call(
        _mlp_kernel,
        out_shape=jax.ShapeDtypeStruct((_NA, _D), jnp.float32),
    )(
        table_a_emb,
        keys_a,
        partial,
        hist,
        W1,
        b1.reshape(1, -1),
        W2,
        b2.reshape(1, -1),
    )
    return out


# speedup vs baseline: 1.1930x; 1.0743x over previous
import functools

import jax
import jax.numpy as jnp
from jax import lax
from jax.experimental import pallas as pl
from jax.experimental.pallas import tpu as pltpu
from jax.experimental.pallas import tpu_sc as plsc

_D = 64
_NB = 100000
_NA = 1024
_BUCKETS = 1024
_CAP = 256
_NC = 2
_NS = 16
_NW = _NC * _NS
_RPW = 3200
_NPAD = _RPW * _NW
_NGRP = _RPW // 16
_HSIZE = 1040
_CROWS = 160
_CHUNKS = _RPW // _CROWS


def _iota16():
    return lax.broadcasted_iota(jnp.int32, (16,), 0)


def _sc1_body(keys_hbm, h_out, lp_out, hist_out,
              keys_v, h_v, lp_v, hist_v, sbufp, sbufn):
    c = lax.axis_index("c")
    s = lax.axis_index("s")
    w = s * _NC + c
    base_row = w * _RPW
    pltpu.sync_copy(keys_hbm.at[pl.ds(base_row * 4, _RPW * 4)], keys_v)
    iota = _iota16()
    zero = jnp.zeros((16,), jnp.int32)

    def zb(i, carry):
        hist_v[pl.ds(i * 16, 16)] = zero
        return carry

    lax.fori_loop(0, _HSIZE // 16, zb, 0)

    def grp(g, carry):
        row0 = g * 16
        ridx = row0 + iota
        ridx4 = ridx * 4
        k = plsc.load_gather(keys_v, [ridx4])
        for col in (1, 2, 3):
            k = k + plsc.load_gather(keys_v, [ridx4 + col])
        h = jnp.bitwise_and(k, _BUCKETS - 1)
        h = jnp.where(base_row + ridx < _NB, h, _BUCKETS)
        h_v[pl.ds(row0, 16)] = h
        hs, lid = plsc.sort_key_val(h, iota)
        sbufp[pl.ds(0, 16)] = hs
        sbufp[pl.ds(1, 16)] = hs
        prev = sbufp[pl.ds(0, 16)]
        sbufn[pl.ds(1, 16)] = hs
        sbufn[pl.ds(0, 16)] = hs
        nxt = sbufn[pl.ds(1, 16)]
        is_start = (hs != prev) | (iota == 0)
        is_last = (hs != nxt) | (iota == 15)
        start_pos = plsc.cummax(jnp.where(is_start, iota, 0))
        rank = iota - start_pos
        cur = plsc.load_gather(hist_v, [hs])
        lp_sorted = cur + rank
        plsc.store_scatter(hist_v, [hs], lp_sorted + 1, mask=is_last)
        plsc.store_scatter(lp_v, [row0 + lid], lp_sorted)
        return carry

    lax.fori_loop(0, _NGRP, grp, 0)
    pltpu.sync_copy(h_v, h_out.at[w])
    pltpu.sync_copy(lp_v, lp_out.at[w])
    pltpu.sync_copy(hist_v, hist_out.at[w])


_HD = _D // 2
_NSLAB = 1


_SLICE = _NA // _NS


def _sc2_body(h_hbm, lp_hbm, base_hbm, emb_hbm, partial_out,
              base_v, h_v, lp_v, ebuf, acc_v, red_v, tmp_v, stage, sem):
    c = lax.axis_index("c")
    s = lax.axis_index("s")
    w = s * _NC + c
    base_row = w * _RPW
    pltpu.sync_copy(h_hbm.at[w], h_v)
    pltpu.sync_copy(lp_hbm.at[w], lp_v)
    pltpu.sync_copy(base_hbm.at[w], base_v)
    zf = jnp.zeros((16,), jnp.float32)

    def grp(g, carry):
        row0 = g * 16
        h = h_v[pl.ds(row0, 16)]
        lpv = lp_v[pl.ds(row0, 16)]
        b = plsc.load_gather(base_v, [h])
        h_v[pl.ds(row0, 16)] = jnp.where(b + lpv < _CAP, h, _BUCKETS)
        return carry

    lax.fori_loop(0, _NGRP, grp, 0)

    for half in range(2):
        def zacc(i, carry):
            acc_v[pl.ds(i * 16, 16)] = zf
            return carry

        lax.fori_loop(0, _HSIZE * _HD // 16, zacc, 0)

        nlive = jnp.maximum(
            jnp.minimum((_NB - base_row) // _CROWS, _CHUNKS), 0)

        def _start(j):
            off = lax.rem(j, 2) * (_CROWS * _D)
            pltpu.async_copy(
                emb_hbm.at[pl.ds((base_row + j * _CROWS) * _D, _CROWS * _D)],
                ebuf.at[pl.ds(off, _CROWS * _D)],
                sem)

        def _wait(j):
            off = lax.rem(j, 2) * (_CROWS * _D)
            pltpu.make_async_copy(
                emb_hbm.at[pl.ds((base_row + j * _CROWS) * _D, _CROWS * _D)],
                ebuf.at[pl.ds(off, _CROWS * _D)],
                sem).wait()

        @pl.when(nlive > 0)
        def _prime():
            _start(0)

        def chunk_body(j, carry):
            @pl.when(j + 1 < nlive)
            def _next():
                _start(j + 1)

            _wait(j)
            off = lax.rem(j, 2) * (_CROWS * _D)

            def grp_add(g, jj):
                tgt16 = h_v[pl.ds(jj * _CROWS + g * 16, 16)]
                for l in range(16):
                    a0 = tgt16[l] * _HD
                    e0 = off + (g * 16 + l) * _D + half * _HD
                    for col in range(_HD // 16):
                        acc_v[pl.ds(a0 + col * 16, 16)] = (
                            acc_v[pl.ds(a0 + col * 16, 16)]
                            + ebuf[pl.ds(e0 + col * 16, 16)])
                return jj

            lax.fori_loop(0, _CROWS // 16, grp_add, j)
            return carry

        lax.fori_loop(0, nlive, chunk_body, 0)

        r0 = s * _SLICE * _HD

        def zred(i, carry):
            red_v[pl.ds(i * 16, 16)] = zf
            return carry

        lax.fori_loop(0, _SLICE * _HD // 16, zred, 0)

        def wave_body(wave, carry):
            @pl.when(s == wave)
            def _pub():
                pltpu.sync_copy(acc_v, stage)

            plsc.subcore_barrier()
            pltpu.sync_copy(stage.at[pl.ds(r0, _SLICE * _HD)], tmp_v)

            def radd(i, carry2):
                sl = pl.ds(i * 16, 16)
                red_v[sl] = red_v[sl] + tmp_v[sl]
                return carry2

            lax.fori_loop(0, _SLICE * _HD // 16, radd, 0)
            plsc.subcore_barrier()
            return carry

        lax.fori_loop(0, _NS, wave_body, 0)

        pltpu.sync_copy(red_v,
                        partial_out.at[half, c, pl.ds(r0, _SLICE * _HD)])


def _prefix_kernel(hist_ref, out_ref):
    x = hist_ref[...].astype(jnp.float32)
    tri = (
        jax.lax.broadcasted_iota(jnp.int32, (_NW, _NW), 0)
        > jax.lax.broadcasted_iota(jnp.int32, (_NW, _NW), 1)
    ).astype(jnp.float32)
    out_ref[...] = jnp.dot(
        tri, x, preferred_element_type=jnp.float32).astype(jnp.int32)


def _mlp_kernel(ta_ref, ka_ref, part_ref, hist_ref,
                w1_ref, b1_ref, w2_ref, b2_ref, out_ref):
    ha = jnp.sum(ka_ref[...], axis=1) % _BUCKETS
    iota_b = lax.broadcasted_iota(jnp.int32, (_NA, _BUCKETS), 1)
    oh = (ha[:, None] == iota_b).astype(jnp.float32)
    bsum = jnp.concatenate(
        [part_ref[0, 0] + part_ref[0, 1],
         part_ref[1, 0] + part_ref[1, 1]], axis=1)
    agg_sum = jnp.dot(oh, bsum, preferred_element_type=jnp.float32)
    hist_f = hist_ref[...][:, :_BUCKETS].astype(jnp.float32)
    cnt_b = jnp.sum(hist_f, axis=0, keepdims=True)
    cnt_a = lax.dot_general(
        oh, cnt_b, (((1,), (1,)), ((), ())),
        preferred_element_type=jnp.float32)
    scale = 1.0 / jnp.maximum(jnp.minimum(cnt_a, float(_CAP)), 1.0)
    agg = agg_sum * scale
    w1 = w1_ref[...]
    x = (
        jnp.dot(ta_ref[...], w1[:_D], preferred_element_type=jnp.float32)
        + jnp.dot(agg, w1[_D:], preferred_element_type=jnp.float32)
        + b1_ref[...]
    )
    x = jnp.maximum(x, 0.0)
    out_ref[...] = (
        jnp.dot(x, w2_ref[...], preferred_element_type=jnp.float32)
        + b2_ref[...]
    )


def kernel(table_a_emb, table_b_emb, keys_a, keys_b, W1, b1, W2, b2):
    mesh = plsc.VectorSubcoreMesh(core_axis_name="c", subcore_axis_name="s")

    sc_params = pltpu.CompilerParams(needs_layout_passes=False)
    sc1 = functools.partial(
        pl.kernel,
        mesh=mesh,
        compiler_params=sc_params,
        out_type=[
            jax.ShapeDtypeStruct((_NW, _RPW), jnp.int32),
            jax.ShapeDtypeStruct((_NW, _RPW), jnp.int32),
            jax.ShapeDtypeStruct((_NW, _HSIZE), jnp.int32),
        ],
        scratch_types=[
            pltpu.VMEM((_RPW * 4,), jnp.int32),
            pltpu.VMEM((_RPW,), jnp.int32),
            pltpu.VMEM((_RPW,), jnp.int32),
            pltpu.VMEM((_HSIZE,), jnp.int32),
            pltpu.VMEM((32,), jnp.int32),
            pltpu.VMEM((32,), jnp.int32),
        ],
    )(_sc1_body)

    keys_pad = jnp.pad(keys_b, ((0, _NPAD - _NB), (0, 0))).reshape(_NPAD * 4)
    h_all, lp_all, hist = sc1(keys_pad)

    base = pl.pallas_call(
        _prefix_kernel,
        out_shape=jax.ShapeDtypeStruct((_NW, _HSIZE), jnp.int32),
    )(hist)

    sc2 = functools.partial(
        pl.kernel,
        mesh=mesh,
        compiler_params=sc_params,
        out_type=[
            jax.ShapeDtypeStruct((2, _NC, _NA * _HD), jnp.float32),
        ],
        scratch_types=[
            pltpu.VMEM((_HSIZE,), jnp.int32),
            pltpu.VMEM((_RPW,), jnp.int32),
            pltpu.VMEM((_RPW,), jnp.int32),
            pltpu.VMEM((2 * _CROWS * _D,), jnp.float32),
            pltpu.VMEM((_HSIZE * _HD,), jnp.float32),
            pltpu.VMEM((_SLICE * _HD,), jnp.float32),
            pltpu.VMEM((_SLICE * _HD,), jnp.float32),
            pltpu.VMEM_SHARED((_HSIZE * _HD,), jnp.float32),
            pltpu.SemaphoreType.DMA,
        ],
    )(_sc2_body)

    (partial,) = sc2(h_all, lp_all, base, table_b_emb.reshape(_NB * _D))
    partial = partial.reshape(2, _NC, _NA, _HD)

    out = pl.pallas_call(
        _mlp_kernel,
        out_shape=jax.ShapeDtypeStruct((_NA, _D), jnp.float32),
    )(
        table_a_emb,
        keys_a,
        partial,
        hist,
        W1,
        b1.reshape(1, -1),
        W2,
        b2.reshape(1, -1),
    )
    return out


# speedup vs baseline: 1.2766x; 1.0700x over previous
import functools

import jax
import jax.numpy as jnp
from jax import lax
from jax.experimental import pallas as pl
from jax.experimental.pallas import tpu as pltpu
from jax.experimental.pallas import tpu_sc as plsc

_D = 64
_NB = 100000
_NA = 1024
_BUCKETS = 1024
_CAP = 256
_NC = 2
_NS = 16
_NW = _NC * _NS
_RPW = 3200
_NPAD = _RPW * _NW
_NGRP = _RPW // 16
_HSIZE = 1040
_CROWS = 160
_CHUNKS = _RPW // _CROWS


def _iota16():
    return lax.broadcasted_iota(jnp.int32, (16,), 0)


def _sc1_body(keys_hbm, h_out, lp_out, hist_out,
              keys_v, h_v, lp_v, hist_v, sbufp, sbufn):
    c = lax.axis_index("c")
    s = lax.axis_index("s")
    w = s * _NC + c
    base_row = w * _RPW
    pltpu.sync_copy(keys_hbm.at[pl.ds(base_row * 4, _RPW * 4)], keys_v)
    iota = _iota16()
    zero = jnp.zeros((16,), jnp.int32)

    def zb(i, carry):
        hist_v[pl.ds(i * 16, 16)] = zero
        return carry

    lax.fori_loop(0, _HSIZE // 16, zb, 0)

    def grp(g, carry):
        row0 = g * 16
        ridx = row0 + iota
        ridx4 = ridx * 4
        k = plsc.load_gather(keys_v, [ridx4])
        for col in (1, 2, 3):
            k = k + plsc.load_gather(keys_v, [ridx4 + col])
        h = jnp.bitwise_and(k, _BUCKETS - 1)
        h = jnp.where(base_row + ridx < _NB, h, _BUCKETS)
        h_v[pl.ds(row0, 16)] = h
        hs, lid = plsc.sort_key_val(h, iota)
        sbufp[pl.ds(0, 16)] = hs
        sbufp[pl.ds(1, 16)] = hs
        prev = sbufp[pl.ds(0, 16)]
        sbufn[pl.ds(1, 16)] = hs
        sbufn[pl.ds(0, 16)] = hs
        nxt = sbufn[pl.ds(1, 16)]
        is_start = (hs != prev) | (iota == 0)
        is_last = (hs != nxt) | (iota == 15)
        start_pos = plsc.cummax(jnp.where(is_start, iota, 0))
        rank = iota - start_pos
        cur = plsc.load_gather(hist_v, [hs])
        lp_sorted = cur + rank
        plsc.store_scatter(hist_v, [hs], lp_sorted + 1, mask=is_last)
        plsc.store_scatter(lp_v, [row0 + lid], lp_sorted)
        return carry

    lax.fori_loop(0, _NGRP, grp, 0)
    pltpu.sync_copy(h_v, h_out.at[w])
    pltpu.sync_copy(lp_v, lp_out.at[w])
    pltpu.sync_copy(hist_v, hist_out.at[w])


_HD = _D // 2
_NSLAB = 1


_SLICE = _NA // _NS


def _sc2_body(h_hbm, lp_hbm, base_hbm, emb_hbm, outa, outb,
              base_v, h_v, lp_v, ebuf, acc_v, tmp4_v, slab, sem):
    c = lax.axis_index("c")
    s = lax.axis_index("s")
    w = s * _NC + c
    base_row = w * _RPW
    pltpu.sync_copy(h_hbm.at[w], h_v)
    pltpu.sync_copy(lp_hbm.at[w], lp_v)
    pltpu.sync_copy(base_hbm.at[w], base_v)
    zf = jnp.zeros((16,), jnp.float32)

    def grp(g, carry):
        row0 = g * 16
        h = h_v[pl.ds(row0, 16)]
        lpv = lp_v[pl.ds(row0, 16)]
        b = plsc.load_gather(base_v, [h])
        h_v[pl.ds(row0, 16)] = jnp.where(b + lpv < _CAP, h, _BUCKETS)
        return carry

    lax.fori_loop(0, _NGRP, grp, 0)

    for half in range(2):
        def zacc(i, carry):
            acc_v[pl.ds(i * 16, 16)] = zf
            return carry

        lax.fori_loop(0, _HSIZE * _HD // 16, zacc, 0)

        nlive = jnp.maximum(
            jnp.minimum((_NB - base_row) // _CROWS, _CHUNKS), 0)

        def _start(j):
            off = lax.rem(j, 2) * (_CROWS * _D)
            pltpu.async_copy(
                emb_hbm.at[pl.ds((base_row + j * _CROWS) * _D, _CROWS * _D)],
                ebuf.at[pl.ds(off, _CROWS * _D)],
                sem)

        def _wait(j):
            off = lax.rem(j, 2) * (_CROWS * _D)
            pltpu.make_async_copy(
                emb_hbm.at[pl.ds((base_row + j * _CROWS) * _D, _CROWS * _D)],
                ebuf.at[pl.ds(off, _CROWS * _D)],
                sem).wait()

        @pl.when(nlive > 0)
        def _prime():
            _start(0)

        def chunk_body(j, carry):
            @pl.when(j + 1 < nlive)
            def _next():
                _start(j + 1)

            _wait(j)
            off = lax.rem(j, 2) * (_CROWS * _D)

            def grp_add(g, jj):
                tgt16 = h_v[pl.ds(jj * _CROWS + g * 16, 16)]
                for l in range(16):
                    a0 = tgt16[l] * _HD
                    e0 = off + (g * 16 + l) * _D + half * _HD
                    for col in range(_HD // 16):
                        acc_v[pl.ds(a0 + col * 16, 16)] = (
                            acc_v[pl.ds(a0 + col * 16, 16)]
                            + ebuf[pl.ds(e0 + col * 16, 16)])
                return jj

            lax.fori_loop(0, _CROWS // 16, grp_add, j)
            return carry

        lax.fori_loop(0, nlive, chunk_body, 0)

        @pl.when(s == _NS - 1)
        def _pub4():
            pltpu.sync_copy(acc_v.at[pl.ds(1020 * _HD, 4 * _HD)], slab)

        plsc.subcore_barrier()

        @pl.when(s == _NS - 2)
        def _fold():
            pltpu.sync_copy(slab, tmp4_v)

            def f4(i, carry):
                acc_v[pl.ds(1020 * _HD + i * 16, 16)] = (
                    acc_v[pl.ds(1020 * _HD + i * 16, 16)]
                    + tmp4_v[pl.ds(i * 16, 16)])
                return carry

            lax.fori_loop(0, 4 * _HD // 16, f4, 0)

        @pl.when(s < _NS - 1)
        def _outa():
            pltpu.sync_copy(acc_v.at[pl.ds(0, _NA * _HD)],
                            outa.at[half, c, s])

        @pl.when(s == _NS - 1)
        def _outb():
            pltpu.sync_copy(acc_v.at[pl.ds(0, 1020 * _HD)], outb.at[half, c])


def _prefix_kernel(hist_ref, out_ref):
    x = hist_ref[...].astype(jnp.float32)
    tri = (
        jax.lax.broadcasted_iota(jnp.int32, (_NW, _NW), 0)
        > jax.lax.broadcasted_iota(jnp.int32, (_NW, _NW), 1)
    ).astype(jnp.float32)
    out_ref[...] = jnp.dot(
        tri, x, preferred_element_type=jnp.float32).astype(jnp.int32)


def _mlp_kernel(ta_ref, ka_ref, pa_ref, pb_ref, hist_ref,
                w1_ref, b1_ref, w2_ref, b2_ref, out_ref):
    ha = jnp.sum(ka_ref[...], axis=1) % _BUCKETS
    iota_b = lax.broadcasted_iota(jnp.int32, (_NA, _BUCKETS), 1)
    oh = (ha[:, None] == iota_b).astype(jnp.float32)
    halves = []
    for h in range(2):
        sh = jnp.sum(jnp.sum(pa_ref[h], axis=0), axis=0)
        t15 = jnp.concatenate(
            [pb_ref[h, 0] + pb_ref[h, 1], jnp.zeros((4, _HD), jnp.float32)],
            axis=0)
        halves.append(sh + t15)
    bsum = jnp.concatenate(halves, axis=1)
    agg_sum = jnp.dot(oh, bsum, preferred_element_type=jnp.float32)
    hist_f = hist_ref[...][:, :_BUCKETS].astype(jnp.float32)
    cnt_b = jnp.sum(hist_f, axis=0, keepdims=True)
    cnt_a = lax.dot_general(
        oh, cnt_b, (((1,), (1,)), ((), ())),
        preferred_element_type=jnp.float32)
    scale = 1.0 / jnp.maximum(jnp.minimum(cnt_a, float(_CAP)), 1.0)
    agg = agg_sum * scale
    w1 = w1_ref[...]
    x = (
        jnp.dot(ta_ref[...], w1[:_D], preferred_element_type=jnp.float32)
        + jnp.dot(agg, w1[_D:], preferred_element_type=jnp.float32)
        + b1_ref[...]
    )
    x = jnp.maximum(x, 0.0)
    out_ref[...] = (
        jnp.dot(x, w2_ref[...], preferred_element_type=jnp.float32)
        + b2_ref[...]
    )


def kernel(table_a_emb, table_b_emb, keys_a, keys_b, W1, b1, W2, b2):
    mesh = plsc.VectorSubcoreMesh(core_axis_name="c", subcore_axis_name="s")

    sc_params = pltpu.CompilerParams(needs_layout_passes=False)
    sc1 = functools.partial(
        pl.kernel,
        mesh=mesh,
        compiler_params=sc_params,
        out_type=[
            jax.ShapeDtypeStruct((_NW, _RPW), jnp.int32),
            jax.ShapeDtypeStruct((_NW, _RPW), jnp.int32),
            jax.ShapeDtypeStruct((_NW, _HSIZE), jnp.int32),
        ],
        scratch_types=[
            pltpu.VMEM((_RPW * 4,), jnp.int32),
            pltpu.VMEM((_RPW,), jnp.int32),
            pltpu.VMEM((_RPW,), jnp.int32),
            pltpu.VMEM((_HSIZE,), jnp.int32),
            pltpu.VMEM((32,), jnp.int32),
            pltpu.VMEM((32,), jnp.int32),
        ],
    )(_sc1_body)

    keys_pad = jnp.pad(keys_b, ((0, _NPAD - _NB), (0, 0))).reshape(_NPAD * 4)
    h_all, lp_all, hist = sc1(keys_pad)

    base = pl.pallas_call(
        _prefix_kernel,
        out_shape=jax.ShapeDtypeStruct((_NW, _HSIZE), jnp.int32),
    )(hist)

    sc2 = functools.partial(
        pl.kernel,
        mesh=mesh,
        compiler_params=sc_params,
    out_type=[
            jax.ShapeDtypeStruct((2, _NC, _NS - 1, _NA * _HD), jnp.float32),
            jax.ShapeDtypeStruct((2, _NC, 1020 * _HD), jnp.float32),
        ],
        scratch_types=[
            pltpu.VMEM((_HSIZE,), jnp.int32),
            pltpu.VMEM((_RPW,), jnp.int32),
            pltpu.VMEM((_RPW,), jnp.int32),
            pltpu.VMEM((2 * _CROWS * _D,), jnp.float32),
            pltpu.VMEM((_HSIZE * _HD,), jnp.float32),
            pltpu.VMEM((4 * _HD,), jnp.float32),
            pltpu.VMEM_SHARED((4 * _HD,), jnp.float32),
            pltpu.SemaphoreType.DMA,
        ],
    )(_sc2_body)

    pa, pb = sc2(h_all, lp_all, base, table_b_emb.reshape(_NB * _D))
    pa = pa.reshape(2, _NC, _NS - 1, _NA, _HD)
    pb = pb.reshape(2, _NC, 1020, _HD)

    out = pl.pallas_call(
        _mlp_kernel,
        out_shape=jax.ShapeDtypeStruct((_NA, _D), jnp.float32),
    )(
        table_a_emb,
        keys_a,
        pa,
        pb,
        hist,
        W1,
        b1.reshape(1, -1),
        W2,
        b2.reshape(1, -1),
    )
    return out
